# baseline (speedup 1.0000x reference)
import functools

import jax
import jax.numpy as jnp
from jax import lax
from jax.experimental import pallas as pl
from jax.experimental.pallas import tpu as pltpu
from jax.experimental.pallas import tpu_sc as plsc

NUM_USERS = 50000
NUM_ITEMS = 45000
NUM_BRANDS = 5000
N_NODES = NUM_USERS + NUM_ITEMS + NUM_BRANDS
N_EDGES = 1600000
EMBED_DIM = 32
DH = EMBED_DIM // 2

ACC_ROWS = 100096
K = 512
NSUB = K // 128
CHUNKS = 196
EDGES_PER_TILE = K * CHUNKS
ROWS_PER_TILE = EDGES_PER_TILE // 128
E_PAD = EDGES_PER_TILE * 16
R_ARR = E_PAD // 128
E_ARR = R_ARR * 128


def _zero16():
    return jnp.zeros((16,), jnp.float32)


def _layer_body(tab, col2, col2b, row2, val, out,
                colv0, colv1, colv2v, rowv0, rowv1, rowv2v,
                valv0, valv1, valv2v, rows0, rows1, rows2v,
                sidx0, sidx1, sidx2v, acc,
                dsem0, dsem1, dsem2, gsem0, gsem1, gsem2,
                ssem0, ssem1, ssem2):
    cid = lax.axis_index("c")
    sid = lax.axis_index("s")
    colv = (colv0, colv1, colv2v)
    rowv = (rowv0, rowv1, rowv2v)
    valv = (valv0, valv1, valv2v)
    rows = (rows0, rows1, rows2v)
    sidx = (sidx0, sidx1, sidx2v)
    dsem = (dsem0, dsem1, dsem2)
    gsem = (gsem0, gsem1, gsem2)
    ssem = (ssem0, ssem1, ssem2)

    def rbase(ci):
        return sid * ROWS_PER_TILE + ci * NSUB

    def fire_idx(q, rb):
        @pl.when(cid == 0)
        def _c0():
            pltpu.async_copy(col2.at[pl.ds(rb, NSUB)], colv[q], dsem[q])

        @pl.when(cid == 1)
        def _c1():
            pltpu.async_copy(col2b.at[pl.ds(rb, NSUB)], colv[q], dsem[q])

        pltpu.async_copy(row2.at[pl.ds(rb, NSUB)], rowv[q], dsem[q])
        pltpu.async_copy(val.at[pl.ds(rb * 128, K)], valv[q], dsem[q])

    def wait_idx(q):
        pltpu.make_async_copy(col2.at[pl.ds(0, NSUB)], colv[q], dsem[q]).wait()
        pltpu.make_async_copy(row2.at[pl.ds(0, NSUB)], rowv[q], dsem[q]).wait()
        pltpu.make_async_copy(val.at[pl.ds(0, K)], valv[q], dsem[q]).wait()

    def fire_g(q):
        for s in range(NSUB):
            pltpu.async_copy(tab.at[colv[q].at[s]],
                             rows[q].at[pl.ds(s * 128, 128)], gsem[q])

    def wait_g(q):
        for _ in range(NSUB):
            pltpu.make_async_copy(tab.at[colv[q].at[0]],
                                  rows[q].at[pl.ds(0, 128)], gsem[q]).wait()

    def fire_s(q):
        for s in range(NSUB):
            pltpu.async_copy(rows[q].at[pl.ds(s * 128, 128)],
                             acc.at[sidx[q].at[s]], ssem[q], add=True)

    def wait_s(q):
        for _ in range(NSUB):
            pltpu.make_async_copy(rows[q].at[pl.ds(0, 128)],
                                  acc.at[sidx[q].at[0]], ssem[q]).wait()

    def compute(q):
        rb_ref, vb_ref, xb_ref, sb_ref = rows[q], valv[q], rowv[q], sidx[q]
        for s in range(NSUB):
            def gbody(g, c, s=s):
                off = g * 16
                vals16 = vb_ref[pl.ds(s * 128 + off, 16)]
                sb_ref[s, pl.ds(off, 16)] = xb_ref[s, pl.ds(off, 16)]
                for j in range(16):
                    e = s * 128 + off + j
                    sp = vals16.at[jnp.full((16,), j, jnp.int32)].get(
                        mode="promise_in_bounds")
                    rb_ref[e, pl.ds(0, 16)] = rb_ref[e, pl.ds(0, 16)] * sp
                return c
            lax.fori_loop(0, 8, gbody, 0)

    def chunk(u, q, qm, first=False, tail=True):
        wait_g(q)
        compute(q)
        fire_s(q)
        if tail:
            fire_idx(qm, rbase(u + 2))
        if not first:
            wait_s(qm)
        if tail:
            wait_idx(qm)
            fire_g(qm)

    def zbody(i, c):
        rows0[i, pl.ds(0, 16)] = _zero16()
        return c
    lax.fori_loop(0, K, zbody, 0)
    zb = pl.multiple_of(sid * 6256, 8)
    for k in range(12):
        pltpu.sync_copy(rows0.at[pl.ds(0, 512)],
                        acc.at[pl.ds(zb + k * 512, 512)])
    pltpu.sync_copy(rows0.at[pl.ds(0, 112)], acc.at[pl.ds(zb + 6144, 112)])
    plsc.subcore_barrier()

    fire_idx(0, rbase(0))
    fire_idx(1, rbase(1))
    wait_idx(0)
    fire_g(0)
    wait_idx(1)
    fire_g(1)

    chunk(0, 0, 2, first=True)
    chunk(1, 1, 0)

    def triple_body(j, c):
        u = 3 * j + 2
        chunk(u, 2, 1)
        chunk(u + 1, 0, 2)
        chunk(u + 2, 1, 0)
        return c
    lax.fori_loop(0, (CHUNKS - 4) // 3, triple_body, 0)

    chunk(CHUNKS - 2, 2, 1, tail=False)
    chunk(CHUNKS - 1, 0, 2, tail=False)
    wait_s(0)

    plsc.subcore_barrier()
    fb = pl.multiple_of(cid * N_NODES + sid * 6256, 8)

    @pl.when(sid < 15)
    def _flush_full():
        pltpu.sync_copy(acc.at[pl.ds(zb, 6256)], out.at[pl.ds(fb, 6256)])

    @pl.when(sid == 15)
    def _flush_last():
        pltpu.sync_copy(acc.at[pl.ds(zb, 6160)], out.at[pl.ds(fb, 6160)])


_layer = functools.partial(
    pl.kernel,
    out_type=jax.ShapeDtypeStruct((2 * N_NODES, DH), jnp.float32),
    mesh=plsc.VectorSubcoreMesh(core_axis_name="c", subcore_axis_name="s"),
    scratch_types=(
        [pltpu.VMEM((NSUB, 128), jnp.int32) for _ in range(3)]
        + [pltpu.VMEM((NSUB, 128), jnp.int32) for _ in range(3)]
        + [pltpu.VMEM((K,), jnp.float32) for _ in range(3)]
        + [pltpu.VMEM((K, DH), jnp.float32) for _ in range(3)]
        + [pltpu.VMEM((NSUB, 128), jnp.int32) for _ in range(3)]
        + [pltpu.VMEM_SHARED((ACC_ROWS, DH), jnp.float32)]
        + [pltpu.SemaphoreType.DMA for _ in range(9)]
    ),
    compiler_params=pltpu.CompilerParams(use_tc_tiling_on_sc=False),
)(_layer_body)


def _split(x):
    return x.reshape(N_NODES, 2, DH).transpose(1, 0, 2).reshape(
        2 * N_NODES, DH)


def kernel(user_embedding, item_embedding, brand_embedding, adj_indices,
           adj_values):
    ego = jnp.concatenate([user_embedding, item_embedding, brand_embedding],
                          axis=0)
    row = adj_indices[0].astype(jnp.int32)
    col = adj_indices[1].astype(jnp.int32)
    pad = E_ARR - N_EDGES
    row = jnp.concatenate([row, jnp.zeros((pad,), jnp.int32)])
    col = jnp.concatenate([col, jnp.zeros((pad,), jnp.int32)])
    val = jnp.concatenate([adj_values, jnp.zeros((pad,), jnp.float32)])
    row2 = row.reshape(R_ARR, 128)
    col2 = col.reshape(R_ARR, 128)
    col2b = col2 + jnp.int32(N_NODES)
    e0s = _split(ego)
    e1s = _layer(e0s, col2, col2b, row2, val)
    e2s = _layer(e1s, col2, col2b, row2, val)
    e3s = _layer(e2s, col2, col2b, row2, val)
    fs = (e0s + e1s + e2s + e3s) * 0.25
    fin = fs.reshape(2, N_NODES, DH).transpose(1, 0, 2).reshape(
        N_NODES, EMBED_DIM)
    return fin[:NUM_USERS], fin[NUM_USERS:NUM_USERS + NUM_ITEMS]
